# baseline (speedup 1.0000x reference)
import functools

import jax
import jax.numpy as jnp
from jax import lax
from jax.experimental import pallas as pl
from jax.experimental.pallas import tpu as pltpu
from jax.experimental.pallas import tpu_sc as plsc

EPS = 1e-5
B, T, V, D = 16, 2048, 8192, 256
NC, NS, L = 2, 16, 16
NW = NC * NS
DPW = D // NW


def _sc_body(emb_hbm, usage_hbm, codes_hbm, out_hbm,
             tab_v, usage_v, codes_v, out_v, sem_t, sem_u, sem_c, sem_o):
    wid = lax.axis_index("s") * NC + lax.axis_index("c")
    d0 = wid * DPW
    tab_cp = pltpu.async_copy(emb_hbm.at[:, pl.ds(d0, DPW)], tab_v, sem_t)
    usage_cp = pltpu.async_copy(usage_hbm, usage_v, sem_u)
    code_cp = pltpu.async_copy(codes_hbm.at[0, 0, :], codes_v.at[pl.ds(0, T)],
                               sem_c)
    tab_cp.wait()
    usage_cp.wait()

    out_cp = [None, None]
    for b in range(B):
        sc = (b % 2) * T
        slot = b % 2
        code_cp.wait()
        if b + 1 < B:
            code_cp = pltpu.async_copy(
                codes_hbm.at[b + 1, 0, :],
                codes_v.at[pl.ds(((b + 1) % 2) * T, T)], sem_c)
        if out_cp[slot] is not None:
            out_cp[slot].wait()

        @plsc.parallel_loop(0, T, step=L, unroll=4)
        def body(i):
            c = codes_v[pl.ds(sc + i, L)]
            u = plsc.load_gather(usage_v, [c])
            inv = 1.0 / jnp.maximum(u, EPS)
            for j in range(DPW):
                jv = jnp.full((L,), j, jnp.int32)
                vals = plsc.load_gather(tab_v, [c, jv])
                out_v[slot, j, pl.ds(i, L)] = vals * inv

        out_cp[slot] = pltpu.async_copy(
            out_v.at[slot], out_hbm.at[b, pl.ds(d0, DPW), :], sem_o)

    out_cp[0].wait()
    out_cp[1].wait()


_sc_gather = functools.partial(
    pl.kernel,
    out_type=jax.ShapeDtypeStruct((B, D, T), jnp.float32),
    mesh=plsc.VectorSubcoreMesh(core_axis_name="c", subcore_axis_name="s"),
    compiler_params=pltpu.CompilerParams(needs_layout_passes=False,
                                         use_tc_tiling_on_sc=False),
    scratch_types=[
        pltpu.VMEM((V, DPW), jnp.float32),
        pltpu.VMEM((V,), jnp.float32),
        pltpu.VMEM((2 * T,), jnp.int32),
        pltpu.VMEM((2, DPW, T), jnp.float32),
        pltpu.SemaphoreType.DMA,
        pltpu.SemaphoreType.DMA,
        pltpu.SemaphoreType.DMA,
        pltpu.SemaphoreType.DMA,
    ],
)(_sc_body)


@jax.jit
def kernel(codes, embedding_sum, cluster_usage):
    return _sc_gather(embedding_sum, cluster_usage, codes)


# speedup vs baseline: 1.3111x; 1.3111x over previous
import functools

import jax
import jax.numpy as jnp
from jax import lax
from jax.experimental import pallas as pl
from jax.experimental.pallas import tpu as pltpu
from jax.experimental.pallas import tpu_sc as plsc

EPS = 1e-5
B, T, V, D = 16, 2048, 8192, 256
NC, NS, L = 2, 16, 16
NW = NC * NS
DPW = D // NW
TR, TC = D // 8, V // 128


def _prep_body(emb_ref, out_ref):
    t = jnp.transpose(emb_ref[...])
    out_ref[...] = t.reshape(TR, 1, 8, 128)


def _prep(emb):
    return pl.pallas_call(
        _prep_body,
        grid=(TC,),
        in_specs=[pl.BlockSpec((128, D), lambda i: (i, 0))],
        out_specs=pl.BlockSpec((TR, 1, 8, 128), lambda i: (0, i, 0, 0)),
        out_shape=jax.ShapeDtypeStruct((TR, TC, 8, 128), jnp.float32),
    )(emb)


def _sc_body(embT4_hbm, usage_hbm, codes_hbm, out_hbm,
             tab_v, usage_v, codes_v, out_v, sem_t, sem_u, sem_c, sem_o):
    wid = lax.axis_index("s") * NC + lax.axis_index("c")
    d0 = wid * DPW
    tab_cp = pltpu.async_copy(embT4_hbm.at[wid], tab_v, sem_t)
    usage_cp = pltpu.async_copy(usage_hbm, usage_v, sem_u)
    code_cp = pltpu.async_copy(codes_hbm.at[0, 0, :], codes_v.at[pl.ds(0, T)],
                               sem_c)
    tab_cp.wait()
    usage_cp.wait()

    out_cp = [None, None]
    for b in range(B):
        sc = (b % 2) * T
        slot = b % 2
        code_cp.wait()
        if b + 1 < B:
            code_cp = pltpu.async_copy(
                codes_hbm.at[b + 1, 0, :],
                codes_v.at[pl.ds(((b + 1) % 2) * T, T)], sem_c)
        if out_cp[slot] is not None:
            out_cp[slot].wait()

        @plsc.parallel_loop(0, T, step=L, unroll=4)
        def body(i):
            c = codes_v[pl.ds(sc + i, L)]
            ch = lax.shift_right_logical(c, 7)
            cl = lax.bitwise_and(c, jnp.int32(127))
            u = plsc.load_gather(usage_v, [c])
            inv = 1.0 / jnp.maximum(u, EPS)
            for j in range(DPW):
                jv = jnp.full((L,), j, jnp.int32)
                vals = plsc.load_gather(tab_v, [ch, jv, cl])
                out_v[slot, j, pl.ds(i, L)] = vals * inv

        out_cp[slot] = pltpu.async_copy(
            out_v.at[slot], out_hbm.at[b, pl.ds(d0, DPW), :], sem_o)

    out_cp[0].wait()
    out_cp[1].wait()


_sc_gather = functools.partial(
    pl.kernel,
    out_type=jax.ShapeDtypeStruct((B, D, T), jnp.float32),
    mesh=plsc.VectorSubcoreMesh(core_axis_name="c", subcore_axis_name="s"),
    compiler_params=pltpu.CompilerParams(needs_layout_passes=False),
    scratch_types=[
        pltpu.VMEM((TC, DPW, 128), jnp.float32),
        pltpu.VMEM((V,), jnp.float32),
        pltpu.VMEM((2 * T,), jnp.int32),
        pltpu.VMEM((2, DPW, T), jnp.float32),
        pltpu.SemaphoreType.DMA,
        pltpu.SemaphoreType.DMA,
        pltpu.SemaphoreType.DMA,
        pltpu.SemaphoreType.DMA,
    ],
)(_sc_body)


@jax.jit
def kernel(codes, embedding_sum, cluster_usage):
    embT4 = _prep(embedding_sum)
    return _sc_gather(embT4, cluster_usage, codes)


# speedup vs baseline: 1.9568x; 1.4924x over previous
import functools

import jax
import jax.numpy as jnp
from jax import lax
from jax.experimental import pallas as pl
from jax.experimental.pallas import tpu as pltpu
from jax.experimental.pallas import tpu_sc as plsc

EPS = 1e-5
B, T, V, D = 16, 2048, 8192, 256
NC, NS, L = 2, 16, 16
NW = NC * NS
DPW = D // NW
TR, TCN = D // 8, V // 128
VB = 1024


def _prep_body(emb_ref, usage_ref, out_ref):
    x = emb_ref[...]
    u = usage_ref[...]
    for k in range(VB // 128):
        inv = 1.0 / jnp.maximum(u[k:k + 1, :], EPS)
        t = jnp.transpose(x[k * 128:(k + 1) * 128, :]) * inv
        out_ref[:, k] = t.reshape(TR, 8, 128)


def _prep(emb, usage2d):
    return pl.pallas_call(
        _prep_body,
        grid=(V // VB,),
        in_specs=[
            pl.BlockSpec((VB, D), lambda i: (i, 0)),
            pl.BlockSpec((VB // 128, 128), lambda i: (i, 0)),
        ],
        out_specs=pl.BlockSpec((TR, VB // 128, 8, 128),
                               lambda i: (0, i, 0, 0)),
        out_shape=jax.ShapeDtypeStruct((TR, TCN, 8, 128), jnp.float32),
    )(emb, usage2d)


def _sc_body(embT4_hbm, codes_hbm, out_hbm,
             tab_v, codes_v, out_v, sem_t, sem_c, sem_o):
    wid = lax.axis_index("s") * NC + lax.axis_index("c")
    d0 = wid * DPW
    tab_cp = pltpu.async_copy(embT4_hbm.at[wid], tab_v, sem_t)
    code_cp = pltpu.async_copy(codes_hbm.at[0, 0, :], codes_v.at[pl.ds(0, T)],
                               sem_c)
    tab_cp.wait()

    out_cp = [None, None]
    for b in range(B):
        sc = (b % 2) * T
        slot = b % 2
        code_cp.wait()
        if b + 1 < B:
            code_cp = pltpu.async_copy(
                codes_hbm.at[b + 1, 0, :],
                codes_v.at[pl.ds(((b + 1) % 2) * T, T)], sem_c)
        if out_cp[slot] is not None:
            out_cp[slot].wait()

        @plsc.parallel_loop(0, T, step=L, unroll=8)
        def body(i):
            c = codes_v[pl.ds(sc + i, L)]
            ch = lax.shift_right_logical(c, 7)
            cl = lax.bitwise_and(c, jnp.int32(127))
            for j in range(DPW):
                jv = jnp.full((L,), j, jnp.int32)
                out_v[slot, j, pl.ds(i, L)] = plsc.load_gather(
                    tab_v, [ch, jv, cl])

        out_cp[slot] = pltpu.async_copy(
            out_v.at[slot], out_hbm.at[b, pl.ds(d0, DPW), :], sem_o)

    out_cp[0].wait()
    out_cp[1].wait()


_sc_gather = functools.partial(
    pl.kernel,
    out_type=jax.ShapeDtypeStruct((B, D, T), jnp.float32),
    mesh=plsc.VectorSubcoreMesh(core_axis_name="c", subcore_axis_name="s"),
    compiler_params=pltpu.CompilerParams(needs_layout_passes=False),
    scratch_types=[
        pltpu.VMEM((TCN, DPW, 128), jnp.float32),
        pltpu.VMEM((2 * T,), jnp.int32),
        pltpu.VMEM((2, DPW, T), jnp.float32),
        pltpu.SemaphoreType.DMA,
        pltpu.SemaphoreType.DMA,
        pltpu.SemaphoreType.DMA,
    ],
)(_sc_body)


@jax.jit
def kernel(codes, embedding_sum, cluster_usage):
    embT4 = _prep(embedding_sum, cluster_usage.reshape(V // 128, 128))
    return _sc_gather(embT4, codes)
